# baseline (speedup 1.0000x reference)
import functools

import jax
import jax.numpy as jnp
from jax import lax
from jax.experimental import pallas as pl
from jax.experimental.pallas import tpu as pltpu
from jax.experimental.pallas import tpu_sc as plsc

V = 100000
D = 70
DP = 80
B = 16384
H = 50

NC, NS = 2, 16
NW = NC * NS
BPW = B // NW
CH = 8
NCHUNK = BPW // CH

BM = 1024
BN = 2048


def _pool_body(table_hbm, idx_hbm, out_hbm, idx_v, rows_v, acc_v, sem):
    wid = lax.axis_index("s") * NC + lax.axis_index("c")
    base_b = wid * BPW

    @pl.loop(0, NCHUNK)
    def _(ci):
        b0 = base_b + ci * CH
        pltpu.sync_copy(idx_hbm.at[pl.ds(b0 * H, CH * H)], idx_v)
        pltpu.async_copy(table_hbm.at[idx_v], rows_v, sem).wait()
        for b in range(CH):
            for c in range(DP // 16):
                def body(j, acc, _b=b, _c=c):
                    return acc + rows_v[_b * H + j, pl.ds(_c * 16, 16)]
                acc = lax.fori_loop(0, H, body, jnp.zeros((16,), jnp.float32))
                acc_v[b, pl.ds(c * 16, 16)] = acc * (1.0 / H)
        pltpu.sync_copy(acc_v, out_hbm.at[pl.ds(b0, CH)])


_pool = functools.partial(
    pl.kernel,
    out_type=jax.ShapeDtypeStruct((B, DP), jnp.float32),
    mesh=plsc.VectorSubcoreMesh(
        core_axis_name="c", subcore_axis_name="s", num_cores=NC, num_subcores=NS
    ),
    scratch_types=[
        pltpu.VMEM((CH * H,), jnp.int32),
        pltpu.VMEM((CH * H, DP), jnp.float32),
        pltpu.VMEM((CH, DP), jnp.float32),
        pltpu.SemaphoreType.DMA,
    ],
    compiler_params=pltpu.CompilerParams(use_tc_tiling_on_sc=False),
)(_pool_body)


def _mm_body(p_ref, w_ref, b_ref, o_ref):
    o_ref[...] = (
        jnp.dot(p_ref[...], w_ref[...], preferred_element_type=jnp.float32)
        + b_ref[...]
    )


def _matmul(p, wt, bias2):
    return pl.pallas_call(
        _mm_body,
        grid=(B // BM, pl.cdiv(V, BN)),
        in_specs=[
            pl.BlockSpec((BM, DP), lambda i, j: (i, 0)),
            pl.BlockSpec((DP, BN), lambda i, j: (0, j)),
            pl.BlockSpec((1, BN), lambda i, j: (0, j)),
        ],
        out_specs=pl.BlockSpec((BM, BN), lambda i, j: (i, j)),
        out_shape=jax.ShapeDtypeStruct((B, V), jnp.float32),
        compiler_params=pltpu.CompilerParams(
            dimension_semantics=("parallel", "arbitrary")
        ),
    )(p, wt, bias2)


def kernel(x, embed_weight, lin_weight, lin_bias):
    table = jnp.pad(embed_weight, ((0, 0), (0, DP - D)))
    idx = x.reshape(-1).astype(jnp.int32)
    pooled = _pool(table, idx)
    p = pooled.astype(jnp.bfloat16)
    wt = jnp.pad(lin_weight, ((0, 0), (0, DP - D))).T.astype(jnp.bfloat16)
    bias2 = lin_bias.reshape(1, V)
    return _matmul(p, wt, bias2)
